# baseline (speedup 1.0000x reference)
import functools

import jax
import jax.numpy as jnp
from jax import lax
from jax.experimental import pallas as pl
from jax.experimental.pallas import tpu as pltpu
from jax.experimental.pallas import tpu_sc as plsc

_NUM_CORES = 2
_NUM_SUBCORES = 16
_NUM_WORKERS = _NUM_CORES * _NUM_SUBCORES

_CHUNK = 200
_NBUF = 4
_INFLIGHT = 3


@jax.jit
def _sc_gather(ids, tab):
  bsz, seq = ids.shape
  dim = tab.shape[1]
  b_per_w = bsz // _NUM_WORKERS
  halves = seq // _CHUNK
  n_chunks = b_per_w * halves
  mesh = plsc.VectorSubcoreMesh(core_axis_name="c", subcore_axis_name="s")

  @functools.partial(
      pl.kernel,
      mesh=mesh,
      compiler_params=pltpu.CompilerParams(use_tc_tiling_on_sc=False),
      out_type=jax.ShapeDtypeStruct((bsz, seq, dim), jnp.float32),
      scratch_types=[
          pltpu.VMEM((b_per_w, seq), jnp.int32),
          *[pltpu.VMEM((_CHUNK, dim), jnp.float32) for _ in range(_NBUF)],
          *[pltpu.SemaphoreType.DMA for _ in range(2 * _NBUF)],
      ],
  )
  def k(ids_hbm, tab_hbm, out_hbm, idx_v, *bufs_and_sems):
    bufs = bufs_and_sems[:_NBUF]
    gsems = bufs_and_sems[_NBUF:2 * _NBUF]
    ssems = bufs_and_sems[2 * _NBUF:]
    wid = lax.axis_index("s") * _NUM_CORES + lax.axis_index("c")
    b0 = wid * b_per_w
    pltpu.sync_copy(ids_hbm.at[pl.ds(b0, b_per_w)], idx_v)

    def idx_ref(t):
      return idx_v.at[t // halves, pl.ds((t % halves) * _CHUNK, _CHUNK)]

    def out_ref(t):
      return out_hbm.at[b0 + t // halves, pl.ds((t % halves) * _CHUNK, _CHUNK)]

    def start_gather(t, kbuf):
      pltpu.async_copy(tab_hbm.at[idx_ref(t)], bufs[kbuf], gsems[kbuf])

    def wait_gather(t, kbuf):
      pltpu.make_async_copy(
          tab_hbm.at[idx_ref(t)], bufs[kbuf], gsems[kbuf]).wait()

    def start_store(t, kbuf):
      pltpu.async_copy(bufs[kbuf], out_ref(t), ssems[kbuf])

    def wait_store(t, kbuf):
      pltpu.make_async_copy(bufs[kbuf], out_ref(t), ssems[kbuf]).wait()

    for t in range(_INFLIGHT):
      start_gather(t, t % _NBUF)

    def body(i, carry):
      for kk in range(_NBUF):
        t = _NBUF * i + kk
        wait_gather(t, kk)
        start_store(t, kk)
        tg = t + _INFLIGHT
        kg = (kk + _INFLIGHT) % _NBUF

        @pl.when(jnp.logical_and(tg >= _NBUF, tg < n_chunks))
        def _():
          wait_store(tg - _NBUF, kg)
          start_gather(tg, kg)

        @pl.when(jnp.logical_and(tg < _NBUF, tg < n_chunks))
        def _():
          start_gather(tg, kg)

      return carry

    lax.fori_loop(0, n_chunks // _NBUF, body, 0)
    for kk in range(_NBUF):
      t = n_chunks - _NBUF + kk
      wait_store(t, kk)

  return k(ids, tab)


def kernel(input_ids, table):
  return _sc_gather(input_ids.astype(jnp.int32), table)


# speedup vs baseline: 1.0055x; 1.0055x over previous
import functools

import jax
import jax.numpy as jnp
from jax import lax
from jax.experimental import pallas as pl
from jax.experimental.pallas import tpu as pltpu
from jax.experimental.pallas import tpu_sc as plsc

_NUM_CORES = 2
_NUM_SUBCORES = 16
_NUM_WORKERS = _NUM_CORES * _NUM_SUBCORES

_CHUNK = 200
_NBUF = 4
_INFLIGHT = 2


@jax.jit
def _sc_gather(ids, tab):
  bsz, seq = ids.shape
  dim = tab.shape[1]
  b_per_w = bsz // _NUM_WORKERS
  halves = seq // _CHUNK
  n_chunks = b_per_w * halves
  mesh = plsc.VectorSubcoreMesh(core_axis_name="c", subcore_axis_name="s")

  @functools.partial(
      pl.kernel,
      mesh=mesh,
      compiler_params=pltpu.CompilerParams(use_tc_tiling_on_sc=False),
      out_type=jax.ShapeDtypeStruct((bsz, seq, dim), jnp.float32),
      scratch_types=[
          pltpu.VMEM((b_per_w, seq), jnp.int32),
          *[pltpu.VMEM((_CHUNK, dim), jnp.float32) for _ in range(_NBUF)],
          *[pltpu.SemaphoreType.DMA for _ in range(2 * _NBUF)],
      ],
  )
  def k(ids_hbm, tab_hbm, out_hbm, idx_v, *bufs_and_sems):
    bufs = bufs_and_sems[:_NBUF]
    gsems = bufs_and_sems[_NBUF:2 * _NBUF]
    ssems = bufs_and_sems[2 * _NBUF:]
    wid = lax.axis_index("s") * _NUM_CORES + lax.axis_index("c")
    b0 = wid * b_per_w
    pltpu.sync_copy(ids_hbm.at[pl.ds(b0, b_per_w)], idx_v)

    def idx_ref(t):
      return idx_v.at[t // halves, pl.ds((t % halves) * _CHUNK, _CHUNK)]

    def out_ref(t):
      return out_hbm.at[b0 + t // halves, pl.ds((t % halves) * _CHUNK, _CHUNK)]

    def start_gather(t, kbuf):
      pltpu.async_copy(tab_hbm.at[idx_ref(t)], bufs[kbuf], gsems[kbuf])

    def wait_gather(t, kbuf):
      pltpu.make_async_copy(
          tab_hbm.at[idx_ref(t)], bufs[kbuf], gsems[kbuf]).wait()

    def start_store(t, kbuf):
      pltpu.async_copy(bufs[kbuf], out_ref(t), ssems[kbuf])

    def wait_store(t, kbuf):
      pltpu.make_async_copy(bufs[kbuf], out_ref(t), ssems[kbuf]).wait()

    for t in range(_INFLIGHT):
      start_gather(t, t % _NBUF)

    def body(i, carry):
      for kk in range(_NBUF):
        t = _NBUF * i + kk
        wait_gather(t, kk)
        start_store(t, kk)
        tg = t + _INFLIGHT
        kg = (kk + _INFLIGHT) % _NBUF

        @pl.when(jnp.logical_and(tg >= _NBUF, tg < n_chunks))
        def _():
          wait_store(tg - _NBUF, kg)
          start_gather(tg, kg)

        @pl.when(jnp.logical_and(tg < _NBUF, tg < n_chunks))
        def _():
          start_gather(tg, kg)

      return carry

    lax.fori_loop(0, n_chunks // _NBUF, body, 0)
    for kk in range(_NBUF):
      t = n_chunks - _NBUF + kk
      wait_store(t, kk)

  return k(ids, tab)


def kernel(input_ids, table):
  return _sc_gather(input_ids.astype(jnp.int32), table)
